# baseline (speedup 1.0000x reference)
import jax
import jax.numpy as jnp
from jax import lax
from jax.experimental import pallas as pl
from jax.experimental.pallas import tpu as pltpu

_N = 8192
_F_IN = 256
_F_OUT = 256
_BM = 512


def _fts_kernel(seq_ref, w_ref, fts_ref):
    fts_ref[...] = lax.dot_general(
        seq_ref[...], w_ref[...],
        (((1,), (1,)), ((), ())),
        preferred_element_type=jnp.float32,
    )


def _agg_kernel(fts_ref, bias_ref, a_ref, adj_ref, out_ref):
    acc = jnp.dot(adj_ref[...], fts_ref[...], preferred_element_type=jnp.float32)
    acc = acc + bias_ref[...]
    a = a_ref[0, 0]
    out_ref[...] = jnp.where(acc >= 0, acc, a * acc)


def kernel(seq, adj, W, bias, prelu_a):
    seq2 = seq.reshape(_N, _F_IN).astype(jnp.float32)
    adj2 = adj.reshape(_N, _N).astype(jnp.float32)
    bias2 = bias.reshape(1, _F_OUT).astype(jnp.float32)
    a2 = prelu_a.reshape(1, 1).astype(jnp.float32)

    fts = pl.pallas_call(
        _fts_kernel,
        out_shape=jax.ShapeDtypeStruct((_N, _F_OUT), jnp.float32),
    )(seq2, W.astype(jnp.float32))

    out = pl.pallas_call(
        _agg_kernel,
        grid=(_N // _BM,),
        in_specs=[
            pl.BlockSpec((_N, _F_OUT), lambda i: (0, 0)),
            pl.BlockSpec((1, _F_OUT), lambda i: (0, 0)),
            pl.BlockSpec((1, 1), lambda i: (0, 0)),
            pl.BlockSpec((_BM, _N), lambda i: (i, 0)),
        ],
        out_specs=pl.BlockSpec((_BM, _F_OUT), lambda i: (i, 0)),
        out_shape=jax.ShapeDtypeStruct((_N, _F_OUT), jnp.float32),
        compiler_params=pltpu.CompilerParams(
            dimension_semantics=("parallel",),
        ),
    )(fts, bias2, a2, adj2)
    return out.reshape(1, _N, _F_OUT)


# speedup vs baseline: 1.0579x; 1.0579x over previous
import jax
import jax.numpy as jnp
from jax import lax
from jax.experimental import pallas as pl
from jax.experimental.pallas import tpu as pltpu

_N = 8192
_F_IN = 256
_F_OUT = 256
_BM = 1024
_BK = 2048
_NK = _N // _BK


def _gcn_kernel(seq_ref, w_ref, bias_ref, a_ref, adj_ref, out_ref, fts_ref, acc_ref):
    i = pl.program_id(0)
    k = pl.program_id(1)

    @pl.when(i == 0)
    def _compute_fts_chunk():
        fts_ref[pl.ds(k * _BK, _BK), :] = lax.dot_general(
            seq_ref[...], w_ref[...],
            (((1,), (1,)), ((), ())),
            preferred_element_type=jnp.float32,
        )

    part = jnp.dot(
        adj_ref[...], fts_ref[pl.ds(k * _BK, _BK), :],
        preferred_element_type=jnp.float32,
    )

    @pl.when(k == 0)
    def _init():
        acc_ref[...] = jnp.zeros_like(acc_ref)

    acc = acc_ref[...] + part

    @pl.when(k < _NK - 1)
    def _carry():
        acc_ref[...] = acc

    @pl.when(k == _NK - 1)
    def _epilogue():
        accb = acc + bias_ref[...]
        a = a_ref[0, 0]
        out_ref[...] = jnp.where(accb >= 0, accb, a * accb)


def kernel(seq, adj, W, bias, prelu_a):
    seq2 = seq.reshape(_N, _F_IN).astype(jnp.float32)
    adj2 = adj.reshape(_N, _N).astype(jnp.float32)
    bias2 = bias.reshape(1, _F_OUT).astype(jnp.float32)
    a2 = prelu_a.reshape(1, 1).astype(jnp.float32)

    grid = (_N // _BM, _NK)
    out = pl.pallas_call(
        _gcn_kernel,
        grid=grid,
        in_specs=[
            pl.BlockSpec((_BK, _F_IN), lambda i, k: (jnp.where(i == 0, k, _NK - 1), 0)),
            pl.BlockSpec((_F_OUT, _F_IN), lambda i, k: (0, 0)),
            pl.BlockSpec((1, _F_OUT), lambda i, k: (0, 0)),
            pl.BlockSpec((1, 1), lambda i, k: (0, 0)),
            pl.BlockSpec((_BM, _BK), lambda i, k: (i, k)),
        ],
        out_specs=pl.BlockSpec((_BM, _F_OUT), lambda i, k: (i, 0)),
        out_shape=jax.ShapeDtypeStruct((_N, _F_OUT), jnp.float32),
        scratch_shapes=[
            pltpu.VMEM((_N, _F_OUT), jnp.float32),
            pltpu.VMEM((_BM, _F_OUT), jnp.float32),
        ],
    )(seq2, W.astype(jnp.float32), bias2, a2, adj2)
    return out.reshape(1, _N, _F_OUT)


# speedup vs baseline: 1.0790x; 1.0199x over previous
import jax
import jax.numpy as jnp
from jax import lax
from jax.experimental import pallas as pl
from jax.experimental.pallas import tpu as pltpu

_N = 8192
_F_IN = 256
_F_OUT = 256
_BM = 256


def _gcn_block_kernel(seq_ref, w_ref, bias_ref, a_ref, adj_ref, out_ref, fts_ref):
    @pl.when(pl.program_id(0) == 0)
    def _compute_fts():
        fts_ref[...] = lax.dot_general(
            seq_ref[...], w_ref[...],
            (((1,), (1,)), ((), ())),
            preferred_element_type=jnp.float32,
        )

    acc = jnp.dot(adj_ref[...], fts_ref[...], preferred_element_type=jnp.float32)
    acc = acc + bias_ref[...]
    a = a_ref[0, 0]
    out_ref[...] = jnp.where(acc >= 0, acc, a * acc)


def kernel(seq, adj, W, bias, prelu_a):
    seq2 = seq.reshape(_N, _F_IN).astype(jnp.float32)
    adj2 = adj.reshape(_N, _N).astype(jnp.float32)
    bias2 = bias.reshape(1, _F_OUT).astype(jnp.float32)
    a2 = prelu_a.reshape(1, 1).astype(jnp.float32)

    grid = (_N // _BM,)
    out = pl.pallas_call(
        _gcn_block_kernel,
        grid=grid,
        in_specs=[
            pl.BlockSpec((_N, _F_IN), lambda i: (0, 0)),
            pl.BlockSpec((_F_OUT, _F_IN), lambda i: (0, 0)),
            pl.BlockSpec((1, _F_OUT), lambda i: (0, 0)),
            pl.BlockSpec((1, 1), lambda i: (0, 0)),
            pl.BlockSpec((_BM, _N), lambda i: (i, 0)),
        ],
        out_specs=pl.BlockSpec((_BM, _F_OUT), lambda i: (i, 0)),
        out_shape=jax.ShapeDtypeStruct((_N, _F_OUT), jnp.float32),
        scratch_shapes=[pltpu.VMEM((_N, _F_OUT), jnp.float32)],
    )(seq2, W.astype(jnp.float32), bias2, a2, adj2)
    return out.reshape(1, _N, _F_OUT)
